# baseline (speedup 1.0000x reference)
import jax
import jax.numpy as jnp
from jax import lax
from jax.experimental import pallas as pl
from jax.experimental.pallas import tpu as pltpu
from jax.experimental.pallas import tpu_sc as plsc

GAZ = 1000001
EMBED = 32
B = 4096
L = 200
W4_ROWS = 250016
BLK = 128

_INFO = plsc.get_sparse_core_info()
NC = _INFO.num_cores
NS = _INFO.num_subcores
NW = NC * NS


def _body(w4_hbm, ids_hbm, out_hbm,
          ida, idb, ia4, ib4, g4a, g4b, ta, tb,
          gsema, gsemb, osema, osemb):
    wid = lax.axis_index("s") * NC + lax.axis_index("c")
    col0 = wid * BLK
    lane = lax.iota(jnp.int32, 16)
    rows = [lane + h * 16 for h in range(8)]

    def fetch(l, ibuf, i4, g4, gsem):
        pltpu.sync_copy(ids_hbm.at[l, pl.ds(col0, BLK)], ibuf)
        for h in range(8):
            i4[pl.ds(h * 16, 16)] = ibuf[pl.ds(h * 16, 16)] >> 2
        pltpu.async_copy(w4_hbm.at[i4], g4, gsem)

    def extract(ibuf, g4, t_out):
        for h in range(8):
            sub = (ibuf[pl.ds(h * 16, 16)] & 3) << 5
            for c in range(EMBED):
                t_out[c, pl.ds(h * 16, 16)] = plsc.load_gather(
                    g4, [rows[h], sub + c])

    def wait_gather(i4, g4, gsem):
        pltpu.make_async_copy(w4_hbm.at[i4], g4, gsem).wait()

    def out_ref(l):
        return out_hbm.at[l, pl.ds(0, EMBED), pl.ds(col0, BLK)]

    def wait_out(t, osem):
        pltpu.make_async_copy(t, out_ref(0), osem).wait()

    fetch(0, ida, ia4, g4a, gsema)

    def loop_body(k, carry):
        l0 = 2 * k
        l1 = 2 * k + 1
        wait_gather(ia4, g4a, gsema)
        fetch(l1, idb, ib4, g4b, gsemb)

        @pl.when(k > 0)
        def _():
            wait_out(ta, osema)
        extract(ida, g4a, ta)
        pltpu.async_copy(ta, out_ref(l0), osema)

        wait_gather(ib4, g4b, gsemb)

        @pl.when(k < L // 2 - 1)
        def _():
            fetch(l0 + 2, ida, ia4, g4a, gsema)

        @pl.when(k > 0)
        def _():
            wait_out(tb, osemb)
        extract(idb, g4b, tb)
        pltpu.async_copy(tb, out_ref(l1), osemb)
        return carry

    lax.fori_loop(0, L // 2, loop_body, 0)
    wait_out(ta, osema)
    wait_out(tb, osemb)


@jax.jit
def kernel(gazetteer_ids, weight):
    w4 = jnp.concatenate(
        [weight, jnp.zeros((63, EMBED), jnp.float32)]).reshape(W4_ROWS, 128)
    ids_t = gazetteer_ids.T
    mesh = plsc.VectorSubcoreMesh(core_axis_name="c", subcore_axis_name="s")
    out_phys = pl.kernel(
        _body,
        out_type=jax.ShapeDtypeStruct((L, EMBED, B), jnp.float32),
        mesh=mesh,
        scratch_types=[
            pltpu.VMEM((BLK,), jnp.int32),
            pltpu.VMEM((BLK,), jnp.int32),
            pltpu.VMEM((BLK,), jnp.int32),
            pltpu.VMEM((BLK,), jnp.int32),
            pltpu.VMEM((BLK, 128), jnp.float32),
            pltpu.VMEM((BLK, 128), jnp.float32),
            pltpu.VMEM((EMBED, BLK), jnp.float32),
            pltpu.VMEM((EMBED, BLK), jnp.float32),
            pltpu.SemaphoreType.DMA,
            pltpu.SemaphoreType.DMA,
            pltpu.SemaphoreType.DMA,
            pltpu.SemaphoreType.DMA,
        ],
        compiler_params=pltpu.CompilerParams(
            use_tc_tiling_on_sc=True, needs_layout_passes=False),
    )(w4, ids_t)
    return out_phys.transpose(2, 0, 1)


# speedup vs baseline: 1.2686x; 1.2686x over previous
import jax
import jax.numpy as jnp
from jax import lax
from jax.experimental import pallas as pl
from jax.experimental.pallas import tpu as pltpu
from jax.experimental.pallas import tpu_sc as plsc

EMBED = 32
B = 4096
L = 200
W4_ROWS = 250000
BLK = 128
NSLOT = 4
LAST_ID = 1000000

_INFO = plsc.get_sparse_core_info()
NC = _INFO.num_cores
NS = _INFO.num_subcores
NW = NC * NS


def _body(w4_hbm, ids_hbm, wlast_hbm, out_hbm,
          ids_v, wlast_v, i4s, g4s, ts, gsems, osems):
    wid = lax.axis_index("s") * NC + lax.axis_index("c")
    col0 = wid * BLK
    lane = lax.iota(jnp.int32, 16)
    rows = [lane + h * 16 for h in range(8)]
    clamp = jnp.full((16,), W4_ROWS - 1, jnp.int32)

    pltpu.sync_copy(ids_hbm.at[pl.ds(0, L), pl.ds(col0, BLK)], ids_v)
    pltpu.sync_copy(wlast_hbm, wlast_v)

    def start_gather(l, s):
        i4, g4 = i4s[s], g4s[s]
        for h in range(8):
            v = ids_v[l, pl.ds(h * 16, 16)]
            i4[pl.ds(h * 16, 16)] = jnp.minimum(v >> 2, clamp)
        pltpu.async_copy(w4_hbm.at[i4], g4, gsems[s])

    def extract(l, s):
        g4, t_out = g4s[s], ts[s]
        any_last = jnp.zeros((16,), jnp.int32)
        for h in range(8):
            v = ids_v[l, pl.ds(h * 16, 16)]
            sub = (v & 3) << 5
            any_last = any_last | (v == LAST_ID).astype(jnp.int32)
            for c in range(EMBED):
                t_out[c, pl.ds(h * 16, 16)] = plsc.load_gather(
                    g4, [rows[h], sub + c])

        @pl.when(lax.reduce_max(any_last, (0,)) > 0)
        def _fixup():
            wl = [wlast_v[pl.ds(0, 16)], wlast_v[pl.ds(16, 16)]]
            for h in range(8):
                v = ids_v[l, pl.ds(h * 16, 16)]
                m = v == LAST_ID
                for c in range(EMBED):
                    cur = t_out[c, pl.ds(h * 16, 16)]
                    rep = lax.broadcast_in_dim(
                        wl[c // 16][c % 16], (16,), ())
                    t_out[c, pl.ds(h * 16, 16)] = jnp.where(m, rep, cur)

    def out_ref(l):
        return out_hbm.at[l, pl.ds(0, EMBED), pl.ds(col0, BLK)]

    for s in range(NSLOT):
        start_gather(s, s)

    def loop_body(k, carry):
        for s in range(NSLOT):
            l = NSLOT * k + s
            pltpu.make_async_copy(w4_hbm.at[i4s[s]], g4s[s], gsems[s]).wait()

            @pl.when(k > 0)
            def _():
                pltpu.make_async_copy(ts[s], out_ref(0), osems[s]).wait()
            extract(l, s)
            pltpu.async_copy(ts[s], out_ref(l), osems[s])

            @pl.when(k < L // NSLOT - 1)
            def _():
                start_gather(l + NSLOT, s)
        return carry

    lax.fori_loop(0, L // NSLOT, loop_body, 0)
    for s in range(NSLOT):
        pltpu.make_async_copy(ts[s], out_ref(0), osems[s]).wait()


@jax.jit
def kernel(gazetteer_ids, weight):
    w4 = weight[:1000000].reshape(W4_ROWS, 128)
    wlast = weight[LAST_ID]
    ids_t = gazetteer_ids.T
    mesh = plsc.VectorSubcoreMesh(core_axis_name="c", subcore_axis_name="s")
    out_phys = pl.kernel(
        lambda w, i, wl, o, idsv, wlv, *rest: _body(
            w, i, wl, o, idsv, wlv,
            list(rest[0:4]), list(rest[4:8]), list(rest[8:12]),
            list(rest[12:16]), list(rest[16:20])),
        out_type=jax.ShapeDtypeStruct((L, EMBED, B), jnp.float32),
        mesh=mesh,
        scratch_types=(
            [pltpu.VMEM((L, BLK), jnp.int32),
             pltpu.VMEM((EMBED,), jnp.float32)]
            + [pltpu.VMEM((BLK,), jnp.int32) for _ in range(NSLOT)]
            + [pltpu.VMEM((BLK, 128), jnp.float32) for _ in range(NSLOT)]
            + [pltpu.VMEM((EMBED, BLK), jnp.float32) for _ in range(NSLOT)]
            + [pltpu.SemaphoreType.DMA for _ in range(2 * NSLOT)]
        ),
        compiler_params=pltpu.CompilerParams(
            use_tc_tiling_on_sc=True, needs_layout_passes=False),
    )(w4, ids_t, wlast)
    return out_phys.transpose(2, 0, 1)


# speedup vs baseline: 1.8036x; 1.4218x over previous
import jax
import jax.numpy as jnp
from jax import lax
from jax.experimental import pallas as pl
from jax.experimental.pallas import tpu as pltpu
from jax.experimental.pallas import tpu_sc as plsc

EMBED = 32
B = 4096
L = 200
W4_ROWS = 250000
BLK = 128
NSLOT = 4
LAST_ID = 1000000

_INFO = plsc.get_sparse_core_info()
NC = _INFO.num_cores
NS = _INFO.num_subcores
NW = NC * NS


def _body(w4_hbm, ids_hbm, wlast_hbm, out_hbm,
          ids_v, wlast_v, i4s, g4s, ts, gsems, osems):
    wid = lax.axis_index("s") * NC + lax.axis_index("c")
    col0 = wid * BLK
    lane = lax.iota(jnp.int32, 16)
    rows = [lane + h * 16 for h in range(8)]
    clamp = jnp.full((16,), W4_ROWS - 1, jnp.int32)

    pltpu.sync_copy(ids_hbm.at[pl.ds(0, L), pl.ds(col0, BLK)], ids_v)
    pltpu.sync_copy(wlast_hbm, wlast_v)

    def start_gather(l, s):
        i4, g4 = i4s[s], g4s[s]

        @plsc.parallel_loop(0, 8, unroll=8)
        def _(h):
            st = h * 16
            v = ids_v[l, pl.ds(st, 16)]
            plsc.store_scatter(i4, [lane + st], jnp.minimum(v >> 2, clamp))

        pltpu.async_copy(w4_hbm.at[i4], g4, gsems[s])

    def extract(l, s):
        g4, t_out = g4s[s], ts[s]
        any_last = jnp.zeros((16,), jnp.int32)
        subs = [None] * 8
        for h in range(8):
            v = ids_v[l, pl.ds(h * 16, 16)]
            subs[h] = (v & 3) << 5
            any_last = any_last | (v == LAST_ID).astype(jnp.int32)
        for h in range(8):
            sub_h, row_h = subs[h], rows[h]

            @plsc.parallel_loop(0, EMBED, unroll=8)
            def _(c):
                vals = plsc.load_gather(g4, [row_h, sub_h + c])
                crow = lax.broadcast_in_dim(c, (16,), ()).astype(jnp.int32)
                plsc.store_scatter(t_out, [crow, row_h], vals)

        @pl.when(lax.reduce_max(any_last, (0,)) > 0)
        def _fixup():
            wl = [wlast_v[pl.ds(0, 16)], wlast_v[pl.ds(16, 16)]]
            for h in range(8):
                v = ids_v[l, pl.ds(h * 16, 16)]
                m = v == LAST_ID
                for c in range(EMBED):
                    cur = t_out[c, pl.ds(h * 16, 16)]
                    rep = lax.broadcast_in_dim(
                        wl[c // 16][c % 16], (16,), ())
                    t_out[c, pl.ds(h * 16, 16)] = jnp.where(m, rep, cur)

    def out_ref(l):
        return out_hbm.at[l, pl.ds(0, EMBED), pl.ds(col0, BLK)]

    for s in range(NSLOT):
        start_gather(s, s)

    def loop_body(k, carry):
        for s in range(NSLOT):
            l = NSLOT * k + s
            pltpu.make_async_copy(w4_hbm.at[i4s[s]], g4s[s], gsems[s]).wait()

            @pl.when(k > 0)
            def _():
                pltpu.make_async_copy(ts[s], out_ref(0), osems[s]).wait()
            extract(l, s)
            pltpu.async_copy(ts[s], out_ref(l), osems[s])

            @pl.when(k < L // NSLOT - 1)
            def _():
                start_gather(l + NSLOT, s)
        return carry

    lax.fori_loop(0, L // NSLOT, loop_body, 0)
    for s in range(NSLOT):
        pltpu.make_async_copy(ts[s], out_ref(0), osems[s]).wait()


@jax.jit
def kernel(gazetteer_ids, weight):
    w4 = weight[:1000000].reshape(W4_ROWS, 128)
    wlast = weight[LAST_ID]
    ids_t = gazetteer_ids.T
    mesh = plsc.VectorSubcoreMesh(core_axis_name="c", subcore_axis_name="s")
    out_phys = pl.kernel(
        lambda w, i, wl, o, idsv, wlv, *rest: _body(
            w, i, wl, o, idsv, wlv,
            list(rest[0:4]), list(rest[4:8]), list(rest[8:12]),
            list(rest[12:16]), list(rest[16:20])),
        out_type=jax.ShapeDtypeStruct((L, EMBED, B), jnp.float32),
        mesh=mesh,
        scratch_types=(
            [pltpu.VMEM((L, BLK), jnp.int32),
             pltpu.VMEM((EMBED,), jnp.float32)]
            + [pltpu.VMEM((BLK,), jnp.int32) for _ in range(NSLOT)]
            + [pltpu.VMEM((BLK, 128), jnp.float32) for _ in range(NSLOT)]
            + [pltpu.VMEM((EMBED, BLK), jnp.float32) for _ in range(NSLOT)]
            + [pltpu.SemaphoreType.DMA for _ in range(2 * NSLOT)]
        ),
        compiler_params=pltpu.CompilerParams(
            use_tc_tiling_on_sc=True, needs_layout_passes=False),
    )(w4, ids_t, wlast)
    return out_phys.transpose(2, 0, 1)


# speedup vs baseline: 1.9299x; 1.0700x over previous
import jax
import jax.numpy as jnp
from jax import lax
from jax.experimental import pallas as pl
from jax.experimental.pallas import tpu as pltpu
from jax.experimental.pallas import tpu_sc as plsc

EMBED = 32
B = 4096
L = 200
W4_ROWS = 250016
LAST_ID = 1000000
BLK = 128
NSLOT = 4
NCHUNK = 7813
KFULL = 244

_INFO = plsc.get_sparse_core_info()
NC = _INFO.num_cores
NS = _INFO.num_subcores
NW = NC * NS


def _body(wt_hbm, ids_hbm, wtail_hbm, out_hbm,
          w4_hbm, ids_v, vs, w4cs, i4s, g4s, ts,
          isems, aosems, gsems, osems, idsem, bsem):
    cid = lax.axis_index("c")
    sid = lax.axis_index("s")
    wid = sid * NC + cid
    col0 = wid * BLK
    lane = lax.iota(jnp.int32, 16)
    rows = [lane + h * 16 for h in range(8)]

    pltpu.async_copy(ids_hbm.at[pl.ds(0, L), pl.ds(col0, BLK)], ids_v, idsem)

    def chunk_of(k):
        return wid + NW * k

    def in_ref(k, s):
        return wt_hbm.at[pl.ds(0, EMBED), pl.ds(chunk_of(k) * 128, 128)]

    def out_w4_ref(k, s):
        return w4_hbm.at[pl.ds(chunk_of(k) * 32, 32), pl.ds(0, 128)]

    def transpose_chunk(s):
        v, w4c = vs[s], w4cs[s]

        @plsc.parallel_loop(0, 32, unroll=8)
        def _(jj):
            base = jj * 4
            jrow = lax.broadcast_in_dim(jj, (16,), ()).astype(jnp.int32)
            for half in range(2):
                cvec = lane + half * 16
                for sub in range(4):
                    vals = plsc.load_gather(
                        v, [cvec, jrow * 4 + sub])
                    plsc.store_scatter(
                        w4c, [jrow, lane + (sub * 32 + half * 16)], vals)

    for s in range(2):
        pltpu.async_copy(in_ref(s, s), vs[s], isems[s])

    def a_loop(j, carry):
        for s in range(2):
            k = 2 * j + s
            pltpu.make_async_copy(in_ref(k, s), vs[s], isems[s]).wait()

            @pl.when(j > 0)
            def _():
                pltpu.make_async_copy(w4cs[s], out_w4_ref(k, s),
                                      aosems[s]).wait()
            transpose_chunk(s)
            pltpu.async_copy(w4cs[s], out_w4_ref(k, s), aosems[s])

            @pl.when(k + 2 < KFULL)
            def _():
                pltpu.async_copy(in_ref(k + 2, s), vs[s], isems[s])
        return carry

    lax.fori_loop(0, KFULL // 2, a_loop, 0)
    for s in range(2):
        pltpu.make_async_copy(w4cs[s], out_w4_ref(0, s), aosems[s]).wait()

    @pl.when(wid < 4)
    def _():
        pltpu.sync_copy(in_ref(KFULL, 0), vs[0])
        transpose_chunk(0)
        pltpu.sync_copy(w4cs[0], out_w4_ref(KFULL, 0))

    @pl.when(wid == 4)
    def _():
        pltpu.sync_copy(wtail_hbm, vs[0])
        transpose_chunk(0)
        pltpu.sync_copy(
            w4cs[0], w4_hbm.at[pl.ds(249984, 32), pl.ds(0, 128)])

    plsc.subcore_barrier()

    @pl.when(sid == 0)
    def _():
        pl.semaphore_signal(bsem, 1, core_index=1 - cid)
        pl.semaphore_wait(bsem, 1)
    plsc.subcore_barrier()

    pltpu.make_async_copy(
        ids_hbm.at[pl.ds(0, L), pl.ds(col0, BLK)], ids_v, idsem).wait()

    def start_gather(l, s):
        i4, g4 = i4s[s], g4s[s]

        @plsc.parallel_loop(0, 8, unroll=8)
        def _(h):
            st = h * 16
            v = ids_v[l, pl.ds(st, 16)]
            plsc.store_scatter(i4, [lane + st], v >> 2)

        pltpu.async_copy(w4_hbm.at[i4], g4, gsems[s])

    def extract(l, s):
        g4, t_out = g4s[s], ts[s]
        subs = [None] * 8
        for h in range(8):
            v = ids_v[l, pl.ds(h * 16, 16)]
            subs[h] = (v & 3) << 5
        for h in range(8):
            sub_h, row_h = subs[h], rows[h]

            @plsc.parallel_loop(0, EMBED, unroll=8)
            def _(c):
                vals = plsc.load_gather(g4, [row_h, sub_h + c])
                crow = lax.broadcast_in_dim(c, (16,), ()).astype(jnp.int32)
                plsc.store_scatter(t_out, [crow, row_h], vals)

    def out_ref(l):
        return out_hbm.at[l, pl.ds(0, EMBED), pl.ds(col0, BLK)]

    for s in range(NSLOT):
        start_gather(s, s)

    def b_loop(k, carry):
        for s in range(NSLOT):
            l = NSLOT * k + s
            pltpu.make_async_copy(w4_hbm.at[i4s[s]], g4s[s], gsems[s]).wait()

            @pl.when(k > 0)
            def _():
                pltpu.make_async_copy(ts[s], out_ref(0), osems[s]).wait()
            extract(l, s)
            pltpu.async_copy(ts[s], out_ref(l), osems[s])

            @pl.when(k < L // NSLOT - 1)
            def _():
                start_gather(l + NSLOT, s)
        return carry

    lax.fori_loop(0, L // NSLOT, b_loop, 0)
    for s in range(NSLOT):
        pltpu.make_async_copy(ts[s], out_ref(0), osems[s]).wait()


@jax.jit
def kernel(gazetteer_ids, weight):
    wt = weight.T
    ids_t = gazetteer_ids.T
    mesh = plsc.VectorSubcoreMesh(core_axis_name="c", subcore_axis_name="s")
    out_phys = pl.kernel(
        lambda wtr, idsr, wlr, outr, *rest: _body(
            wtr, idsr, wlr, outr, rest[0], rest[1],
            list(rest[2:4]), list(rest[4:6]), list(rest[6:10]),
            list(rest[10:14]), list(rest[14:18]),
            list(rest[18:20]), list(rest[20:22]), list(rest[22:26]),
            list(rest[26:30]), rest[30], rest[31]),
        out_type=jax.ShapeDtypeStruct((L, EMBED, B), jnp.float32),
        mesh=mesh,
        scratch_types=(
            [pltpu.HBM((W4_ROWS, 128), jnp.float32),
             pltpu.VMEM((L, BLK), jnp.int32)]
            + [pltpu.VMEM((EMBED, 128), jnp.float32) for _ in range(2)]
            + [pltpu.VMEM((32, 128), jnp.float32) for _ in range(2)]
            + [pltpu.VMEM((BLK,), jnp.int32) for _ in range(NSLOT)]
            + [pltpu.VMEM((BLK, 128), jnp.float32) for _ in range(NSLOT)]
            + [pltpu.VMEM((EMBED, BLK), jnp.float32) for _ in range(NSLOT)]
            + [pltpu.SemaphoreType.DMA for _ in range(2)]
            + [pltpu.SemaphoreType.DMA for _ in range(2)]
            + [pltpu.SemaphoreType.DMA for _ in range(NSLOT)]
            + [pltpu.SemaphoreType.DMA for _ in range(NSLOT)]
            + [pltpu.SemaphoreType.DMA, pltpu.SemaphoreType.REGULAR]
        ),
        compiler_params=pltpu.CompilerParams(
            use_tc_tiling_on_sc=True, needs_layout_passes=False),
    )(wt, ids_t, jnp.pad(weight[999936:].T, ((0, 0), (0, 63))))
    return out_phys.transpose(2, 0, 1)


# speedup vs baseline: 1.9392x; 1.0048x over previous
import jax
import jax.numpy as jnp
from jax import lax
from jax.experimental import pallas as pl
from jax.experimental.pallas import tpu as pltpu
from jax.experimental.pallas import tpu_sc as plsc

EMBED = 32
B = 4096
L = 200
W4_ROWS = 250016
LAST_ID = 1000000
BLK = 128
NSLOT = 4
NCHUNK = 7813
KFULL = 244

_INFO = plsc.get_sparse_core_info()
NC = _INFO.num_cores
NS = _INFO.num_subcores
NW = NC * NS


def _body(wt_hbm, ids_hbm, wtail_hbm, out_hbm,
          w4_hbm, ids_v, vs, w4cs, i4s, g4s, ts,
          isems, aosems, gsems, osems, idsem, bsem):
    cid = lax.axis_index("c")
    sid = lax.axis_index("s")
    wid = sid * NC + cid
    col0 = wid * BLK
    lane = lax.iota(jnp.int32, 16)
    rows = [lane + h * 16 for h in range(8)]

    pltpu.async_copy(ids_hbm.at[pl.ds(0, L), pl.ds(col0, BLK)], ids_v, idsem)

    def chunk_of(k):
        return wid + NW * k

    def in_ref(k, s):
        return wt_hbm.at[pl.ds(0, EMBED), pl.ds(chunk_of(k) * 128, 128)]

    def out_w4_ref(k, s):
        return w4_hbm.at[pl.ds(chunk_of(k) * 32, 32), pl.ds(0, 128)]

    rowidx = [(lane + j0 * 16) >> 2 for j0 in range(8)]
    colbase = [((lane + j0 * 16) & 3) * 32 for j0 in range(8)]

    def transpose_chunk(s):
        v, w4c = vs[s], w4cs[s]
        for j0 in range(8):
            ri, cb = rowidx[j0], colbase[j0]

            @plsc.parallel_loop(0, EMBED, unroll=8)
            def _(c):
                vals = v[c, pl.ds(j0 * 16, 16)]
                plsc.store_scatter(w4c, [ri, cb + c], vals)

    for s in range(2):
        pltpu.async_copy(in_ref(s, s), vs[s], isems[s])

    def a_loop(j, carry):
        for s in range(2):
            k = 2 * j + s
            pltpu.make_async_copy(in_ref(k, s), vs[s], isems[s]).wait()

            @pl.when(j > 0)
            def _():
                pltpu.make_async_copy(w4cs[s], out_w4_ref(k, s),
                                      aosems[s]).wait()
            transpose_chunk(s)
            pltpu.async_copy(w4cs[s], out_w4_ref(k, s), aosems[s])

            @pl.when(k + 2 < KFULL)
            def _():
                pltpu.async_copy(in_ref(k + 2, s), vs[s], isems[s])
        return carry

    lax.fori_loop(0, KFULL // 2, a_loop, 0)
    for s in range(2):
        pltpu.make_async_copy(w4cs[s], out_w4_ref(0, s), aosems[s]).wait()

    @pl.when(wid < 4)
    def _():
        pltpu.sync_copy(in_ref(KFULL, 0), vs[0])
        transpose_chunk(0)
        pltpu.sync_copy(w4cs[0], out_w4_ref(KFULL, 0))

    @pl.when(wid == 4)
    def _():
        pltpu.sync_copy(wtail_hbm, vs[0])
        transpose_chunk(0)
        pltpu.sync_copy(
            w4cs[0], w4_hbm.at[pl.ds(249984, 32), pl.ds(0, 128)])

    plsc.subcore_barrier()

    @pl.when(sid == 0)
    def _():
        pl.semaphore_signal(bsem, 1, core_index=1 - cid)
        pl.semaphore_wait(bsem, 1)
    plsc.subcore_barrier()

    pltpu.make_async_copy(
        ids_hbm.at[pl.ds(0, L), pl.ds(col0, BLK)], ids_v, idsem).wait()

    def start_gather(l, s):
        i4, g4 = i4s[s], g4s[s]

        @plsc.parallel_loop(0, 8, unroll=8)
        def _(h):
            st = h * 16
            v = ids_v[l, pl.ds(st, 16)]
            plsc.store_scatter(i4, [lane + st], v >> 2)

        pltpu.async_copy(w4_hbm.at[i4], g4, gsems[s])

    def extract(l, s):
        g4, t_out = g4s[s], ts[s]
        subs = [None] * 8
        for h in range(8):
            v = ids_v[l, pl.ds(h * 16, 16)]
            subs[h] = (v & 3) << 5
        for h in range(8):
            sub_h, row_h = subs[h], rows[h]

            @plsc.parallel_loop(0, EMBED, unroll=8)
            def _(c):
                vals = plsc.load_gather(g4, [row_h, sub_h + c])
                crow = lax.broadcast_in_dim(c, (16,), ()).astype(jnp.int32)
                plsc.store_scatter(t_out, [crow, row_h], vals)

    def out_ref(l):
        return out_hbm.at[l, pl.ds(0, EMBED), pl.ds(col0, BLK)]

    for s in range(NSLOT):
        start_gather(s, s)

    def b_loop(k, carry):
        for s in range(NSLOT):
            l = NSLOT * k + s
            pltpu.make_async_copy(w4_hbm.at[i4s[s]], g4s[s], gsems[s]).wait()

            @pl.when(k > 0)
            def _():
                pltpu.make_async_copy(ts[s], out_ref(0), osems[s]).wait()
            extract(l, s)
            pltpu.async_copy(ts[s], out_ref(l), osems[s])

            @pl.when(k < L // NSLOT - 1)
            def _():
                start_gather(l + NSLOT, s)
        return carry

    lax.fori_loop(0, L // NSLOT, b_loop, 0)
    for s in range(NSLOT):
        pltpu.make_async_copy(ts[s], out_ref(0), osems[s]).wait()


@jax.jit
def kernel(gazetteer_ids, weight):
    wt = weight.T
    ids_t = gazetteer_ids.T
    mesh = plsc.VectorSubcoreMesh(core_axis_name="c", subcore_axis_name="s")
    out_phys = pl.kernel(
        lambda wtr, idsr, wlr, outr, *rest: _body(
            wtr, idsr, wlr, outr, rest[0], rest[1],
            list(rest[2:4]), list(rest[4:6]), list(rest[6:10]),
            list(rest[10:14]), list(rest[14:18]),
            list(rest[18:20]), list(rest[20:22]), list(rest[22:26]),
            list(rest[26:30]), rest[30], rest[31]),
        out_type=jax.ShapeDtypeStruct((L, EMBED, B), jnp.float32),
        mesh=mesh,
        scratch_types=(
            [pltpu.HBM((W4_ROWS, 128), jnp.float32),
             pltpu.VMEM((L, BLK), jnp.int32)]
            + [pltpu.VMEM((EMBED, 128), jnp.float32) for _ in range(2)]
            + [pltpu.VMEM((32, 128), jnp.float32) for _ in range(2)]
            + [pltpu.VMEM((BLK,), jnp.int32) for _ in range(NSLOT)]
            + [pltpu.VMEM((BLK, 128), jnp.float32) for _ in range(NSLOT)]
            + [pltpu.VMEM((EMBED, BLK), jnp.float32) for _ in range(NSLOT)]
            + [pltpu.SemaphoreType.DMA for _ in range(2)]
            + [pltpu.SemaphoreType.DMA for _ in range(2)]
            + [pltpu.SemaphoreType.DMA for _ in range(NSLOT)]
            + [pltpu.SemaphoreType.DMA for _ in range(NSLOT)]
            + [pltpu.SemaphoreType.DMA, pltpu.SemaphoreType.REGULAR]
        ),
        compiler_params=pltpu.CompilerParams(
            use_tc_tiling_on_sc=True, needs_layout_passes=False),
    )(wt, ids_t, jnp.pad(weight[999936:].T, ((0, 0), (0, 63))))
    return out_phys.transpose(2, 0, 1)
